# Initial kernel scaffold; baseline (speedup 1.0000x reference)
import jax
import jax.numpy as jnp
from jax.experimental import pallas as pl


def kernel(x_branch, x_branch_index, x_trunk, g1_W, g1_as, g1_ad, g1_b, g2_W, g2_as, g2_ad, g2_b, g3_W, g3_as, g3_ad, g3_b, g4_W, g4_as, g4_ad, g4_b, fc_W, fc_b, out_W, out_b, t1_W, t1_b, t2_W, t2_b, t3_W, t3_b, t4_W, t4_b, bias):
    raise NotImplementedError("write your pallas kernel here")


# speedup vs baseline: 1.0000x; 1.0000x over previous
import jax
import jax.numpy as jnp
from jax.experimental import pallas as pl
from jax.experimental.pallas import tpu as pltpu


def _gat_conv(x, src_all, dst_all, valid, W, a_s, a_d, b, concat):
    N = x.shape[0]
    H, C = a_s.shape
    h = (x @ W).reshape(N, H, C)
    asrc = (h * a_s[None, :, :]).sum(-1)
    adst = (h * a_d[None, :, :]).sum(-1)
    alpha = asrc[src_all] + adst[dst_all]
    alpha = jax.nn.leaky_relu(alpha, 0.2)
    amax = jax.ops.segment_max(alpha, dst_all, num_segments=N)
    ex = jnp.exp(alpha - amax[dst_all]) * valid[:, None]
    den = jax.ops.segment_sum(ex, dst_all, num_segments=N)
    coef = ex / (den[dst_all] + 1e-16)
    out = jax.ops.segment_sum(h[src_all] * coef[:, :, None], dst_all, num_segments=N)
    if concat:
        out = out.reshape(N, H * C)
    else:
        out = out.mean(axis=1)
    return out + b


def _tail_body(x4_ref, fc_W_ref, fc_b_ref, out_W_ref, out_b_ref,
               x_trunk_ref, t1_W_ref, t1_b_ref, t2_W_ref, t2_b_ref,
               t3_W_ref, t3_b_ref, t4_W_ref, t4_b_ref, bias_ref, o_ref):
    x5 = jnp.dot(x4_ref[...], fc_W_ref[...],
                 preferred_element_type=jnp.float32) + fc_b_ref[...]
    bout = jnp.dot(x5, out_W_ref[...],
                   preferred_element_type=jnp.float32) + out_b_ref[...]
    t = jax.nn.relu(jnp.dot(x_trunk_ref[...], t1_W_ref[...],
                            preferred_element_type=jnp.float32) + t1_b_ref[...])
    t = jax.nn.relu(jnp.dot(t, t2_W_ref[...],
                            preferred_element_type=jnp.float32) + t2_b_ref[...])
    t = jax.nn.relu(jnp.dot(t, t3_W_ref[...],
                            preferred_element_type=jnp.float32) + t3_b_ref[...])
    t = jax.nn.relu(jnp.dot(t, t4_W_ref[...],
                            preferred_element_type=jnp.float32) + t4_b_ref[...])
    o_ref[...] = jnp.dot(bout, t.T, preferred_element_type=jnp.float32) + bias_ref[0]


def _tail(x4, fc_W, fc_b, out_W, out_b, x_trunk,
          t1_W, t1_b, t2_W, t2_b, t3_W, t3_b, t4_W, t4_b, bias):
    N = x4.shape[0]
    T = x_trunk.shape[0]
    BLK = 2000
    grid = (N // BLK,)
    full = lambda shape: pl.BlockSpec(shape, lambda i: (0, 0))
    return pl.pallas_call(
        _tail_body,
        grid=grid,
        in_specs=[
            pl.BlockSpec((BLK, 128), lambda i: (i, 0)),
            full(fc_W.shape), pl.BlockSpec((256,), lambda i: (0,)),
            full(out_W.shape), pl.BlockSpec((128,), lambda i: (0,)),
            full(x_trunk.shape),
            full(t1_W.shape), pl.BlockSpec((256,), lambda i: (0,)),
            full(t2_W.shape), pl.BlockSpec((256,), lambda i: (0,)),
            full(t3_W.shape), pl.BlockSpec((256,), lambda i: (0,)),
            full(t4_W.shape), pl.BlockSpec((128,), lambda i: (0,)),
            pl.BlockSpec((1,), lambda i: (0,)),
        ],
        out_specs=pl.BlockSpec((BLK, T), lambda i: (i, 0)),
        out_shape=jax.ShapeDtypeStruct((N, T), jnp.float32),
    )(x4, fc_W, fc_b, out_W, out_b, x_trunk,
      t1_W, t1_b, t2_W, t2_b, t3_W, t3_b, t4_W, t4_b, bias)


def kernel(x_branch, x_branch_index, x_trunk, g1_W, g1_as, g1_ad, g1_b,
           g2_W, g2_as, g2_ad, g2_b, g3_W, g3_as, g3_ad, g3_b,
           g4_W, g4_as, g4_ad, g4_b, fc_W, fc_b, out_W, out_b,
           t1_W, t1_b, t2_W, t2_b, t3_W, t3_b, t4_W, t4_b, bias):
    N = x_branch.shape[0]
    src = x_branch_index[0]
    dst = x_branch_index[1]
    loop = jnp.arange(N, dtype=src.dtype)
    src_all = jnp.concatenate([src, loop])
    dst_all = jnp.concatenate([dst, loop])
    valid = jnp.concatenate([(src != dst), jnp.ones((N,), dtype=bool)]).astype(jnp.float32)
    x = _gat_conv(x_branch, src_all, dst_all, valid, g1_W, g1_as, g1_ad, g1_b, True)
    x = jax.nn.relu(x)
    x = _gat_conv(x, src_all, dst_all, valid, g2_W, g2_as, g2_ad, g2_b, True)
    x = jax.nn.relu(x)
    x = _gat_conv(x, src_all, dst_all, valid, g3_W, g3_as, g3_ad, g3_b, True)
    x = jax.nn.relu(x)
    x = _gat_conv(x, src_all, dst_all, valid, g4_W, g4_as, g4_ad, g4_b, False)
    return _tail(x, fc_W, fc_b, out_W, out_b, x_trunk,
                 t1_W, t1_b, t2_W, t2_b, t3_W, t3_b, t4_W, t4_b, bias)


# speedup vs baseline: 17.0959x; 17.0952x over previous
import functools

import jax
import jax.numpy as jnp
from jax import lax
from jax.experimental import pallas as pl
from jax.experimental.pallas import tpu as pltpu
from jax.experimental.pallas import tpu_sc as plsc

N = 10000
E = 320000
EN = E + N
E_PAD = 330240
TRASH = N
N_ACC = N + 16
ROWS = 10112
RPT = ROWS // 16
K1 = 48
K4 = 24


def _mesh():
    return plsc.VectorSubcoreMesh(core_axis_name="c", subcore_axis_name="s")


def _zero_rows(z_buf, shared, row0, nrows):
    D = shared.shape[1]
    rpc = z_buf.shape[0]

    def zb(r, _):
        for j in range(D // 16):
            z_buf[r, pl.ds(j * 16, 16)] = jnp.zeros((16,), jnp.float32)
        return 0

    lax.fori_loop(0, rpc, zb, 0)
    for j in range(nrows // rpc):
        pltpu.sync_copy(z_buf, shared.at[pl.ds(row0 + j * rpc, rpc)])


def _attn_pass(src_all, dst_all, asrc, adc):
    mesh = _mesh()
    n_chunks = (E_PAD // 2) // 16 // K1

    @functools.partial(
        pl.kernel, mesh=mesh,
        out_type=(jax.ShapeDtypeStruct((E_PAD, 16), jnp.float32),
                  jax.ShapeDtypeStruct((2, ROWS, 16), jnp.float32)),
        scratch_types=[
            pltpu.VMEM((K1,), jnp.int32),
            pltpu.VMEM((K1,), jnp.int32),
            pltpu.VMEM((K1, 128), jnp.float32),
            pltpu.VMEM((K1, 128), jnp.float32),
            pltpu.VMEM((K1, 16), jnp.float32),
            pltpu.VMEM((8, 16), jnp.float32),
            pltpu.VMEM_SHARED((ROWS, 16), jnp.float32),
            pltpu.SemaphoreType.DMA,
        ],
    )
    def k(src_hbm, dst_hbm, asrc_hbm, adc_hbm, ex_hbm, den_hbm,
          idx_s, idx_d, as_buf, adc_buf, ex_buf, zd_buf, den_sh, sem):
        cid = lax.axis_index("c")
        sid = lax.axis_index("s")
        _zero_rows(zd_buf, den_sh, sid * RPT, RPT)
        plsc.subcore_barrier()
        tile_base = cid * (E_PAD // 2) + sid * (E_PAD // 32)

        def edge(e, _):
            z = as_buf[e, pl.ds(0, 16)] + adc_buf[e, pl.ds(0, 16)]
            al = jnp.where(z >= 0.0, z, 0.2 * z)
            ex_buf[e] = jnp.exp(al - adc_buf[e, pl.ds(16, 16)])
            return 0

        def chunk(ch, _):
            base = tile_base + ch * K1
            pltpu.sync_copy(src_hbm.at[pl.ds(base, K1)], idx_s)
            pltpu.sync_copy(dst_hbm.at[pl.ds(base, K1)], idx_d)
            ga = pltpu.async_copy(asrc_hbm.at[idx_s], as_buf, sem)
            gd = pltpu.async_copy(adc_hbm.at[idx_d], adc_buf, sem)
            ga.wait()
            gd.wait()
            lax.fori_loop(0, K1, edge, 0)
            pltpu.sync_copy(ex_buf, ex_hbm.at[pl.ds(base, K1)])
            pltpu.sync_copy(ex_buf, den_sh.at[idx_d], add=True)
            return 0

        lax.fori_loop(0, n_chunks, chunk, 0)
        plsc.subcore_barrier()
        pltpu.sync_copy(
            den_sh.at[pl.ds(sid * RPT, RPT)],
            den_hbm.at[cid].at[pl.ds(sid * RPT, RPT)])

    return k(src_all, dst_all, asrc, adc)


def _agg_pass(src_all, dst_all, ex, den128, h_lo, h_hi):
    mesh = _mesh()
    n_chunks = E_PAD // 16 // K1

    @functools.partial(
        pl.kernel, mesh=mesh,
        out_type=jax.ShapeDtypeStruct((2, ROWS, 128), jnp.float32),
        scratch_types=[
            pltpu.VMEM((K1,), jnp.int32),
            pltpu.VMEM((K1,), jnp.int32),
            pltpu.VMEM((K1, 16), jnp.float32),
            pltpu.VMEM((K1, 128), jnp.float32),
            pltpu.VMEM((K1, 128), jnp.float32),
            pltpu.VMEM((K1, 128), jnp.float32),
            pltpu.VMEM((8, 128), jnp.float32),
            pltpu.VMEM_SHARED((ROWS, 128), jnp.float32),
            pltpu.SemaphoreType.DMA,
        ],
    )
    def k(src_hbm, dst_hbm, ex_hbm, den_hbm, hlo_hbm, hhi_hbm,
          agg_hbm, idx_s, idx_d, ex_buf, d_buf, h_buf, ob_buf, zo_buf,
          out_sh, sem):
        cid = lax.axis_index("c")
        sid = lax.axis_index("s")
        _zero_rows(zo_buf, out_sh, sid * RPT, RPT)
        plsc.subcore_barrier()
        tile_base = sid * (E_PAD // 16)

        def make_edge(ho):
            def edge(e, _):
                den = (d_buf[e, pl.ds(0, 16)] + d_buf[e, pl.ds(16, 16)]
                       + jnp.float32(1e-16))
                coef = ex_buf[e] / den
                for h in range(8):
                    bc = lax.broadcast(coef[ho + h], (16,))
                    ob_buf[e, pl.ds(h * 16, 16)] = (
                        bc * h_buf[e, pl.ds(h * 16, 16)])
                return 0
            return edge

        def chunk(ch, _):
            base = tile_base + ch * K1
            pltpu.sync_copy(src_hbm.at[pl.ds(base, K1)], idx_s)
            pltpu.sync_copy(dst_hbm.at[pl.ds(base, K1)], idx_d)
            pltpu.sync_copy(ex_hbm.at[pl.ds(base, K1)], ex_buf)
            g0 = pltpu.async_copy(den_hbm.at[idx_d], d_buf, sem)

            @pl.when(cid == 0)
            def _():
                pltpu.async_copy(hlo_hbm.at[idx_s], h_buf, sem).wait()

            @pl.when(cid != 0)
            def _():
                pltpu.async_copy(hhi_hbm.at[idx_s], h_buf, sem).wait()

            g0.wait()

            @pl.when(cid == 0)
            def _():
                lax.fori_loop(0, K1, make_edge(0), 0)

            @pl.when(cid != 0)
            def _():
                lax.fori_loop(0, K1, make_edge(8), 0)

            pltpu.sync_copy(ob_buf, out_sh.at[idx_d], add=True)
            return 0

        lax.fori_loop(0, n_chunks, chunk, 0)
        plsc.subcore_barrier()
        pltpu.sync_copy(
            out_sh.at[pl.ds(sid * RPT, RPT)],
            agg_hbm.at[cid].at[pl.ds(sid * RPT, RPT)])

    return k(src_all, dst_all, ex, den128, h_lo, h_hi)


def _agg4_pass(src_all, dst_all, ex, den128, h_lo, h_hi):
    mesh = _mesh()
    n_chunks = E_PAD // 16 // K4

    @functools.partial(
        pl.kernel, mesh=mesh,
        out_type=jax.ShapeDtypeStruct((2, ROWS, 128), jnp.float32),
        scratch_types=[
            pltpu.VMEM((K4,), jnp.int32),
            pltpu.VMEM((K4,), jnp.int32),
            pltpu.VMEM((K4, 16), jnp.float32),
            pltpu.VMEM((K4, 128), jnp.float32),
            pltpu.VMEM((K4, 1024), jnp.float32),
            pltpu.VMEM((K4, 128), jnp.float32),
            pltpu.VMEM((8, 128), jnp.float32),
            pltpu.VMEM_SHARED((ROWS, 128), jnp.float32),
            pltpu.SemaphoreType.DMA,
        ],
    )
    def k(src_hbm, dst_hbm, ex_hbm, den_hbm, hlo_hbm, hhi_hbm,
          agg_hbm, idx_s, idx_d, ex_buf, d_buf, h_buf,
          ob_buf, zo_buf, out_sh, sem):
        cid = lax.axis_index("c")
        sid = lax.axis_index("s")
        _zero_rows(zo_buf, out_sh, sid * RPT, RPT)
        plsc.subcore_barrier()
        tile_base = sid * (E_PAD // 16)

        def make_edge(ho):
            def edge(e, _):
                den = (d_buf[e, pl.ds(0, 16)] + d_buf[e, pl.ds(16, 16)]
                       + jnp.float32(1e-16))
                coef = ex_buf[e] / den
                acc = [jnp.zeros((16,), jnp.float32) for _ in range(8)]
                for h in range(8):
                    bc = lax.broadcast(coef[ho + h], (16,))
                    for j in range(8):
                        acc[j] = acc[j] + bc * h_buf[
                            e, pl.ds(h * 128 + j * 16, 16)]
                for j in range(8):
                    ob_buf[e, pl.ds(j * 16, 16)] = acc[j]
                return 0
            return edge

        def chunk(ch, _):
            base = tile_base + ch * K4
            pltpu.sync_copy(src_hbm.at[pl.ds(base, K4)], idx_s)
            pltpu.sync_copy(dst_hbm.at[pl.ds(base, K4)], idx_d)
            pltpu.sync_copy(ex_hbm.at[pl.ds(base, K4)], ex_buf)
            g0 = pltpu.async_copy(den_hbm.at[idx_d], d_buf, sem)

            @pl.when(cid == 0)
            def _():
                pltpu.async_copy(hlo_hbm.at[idx_s], h_buf, sem).wait()

            @pl.when(cid != 0)
            def _():
                pltpu.async_copy(hhi_hbm.at[idx_s], h_buf, sem).wait()

            g0.wait()

            @pl.when(cid == 0)
            def _():
                lax.fori_loop(0, K4, make_edge(0), 0)

            @pl.when(cid != 0)
            def _():
                lax.fori_loop(0, K4, make_edge(8), 0)

            pltpu.sync_copy(ob_buf, out_sh.at[idx_d], add=True)
            return 0

        lax.fori_loop(0, n_chunks, chunk, 0)
        plsc.subcore_barrier()
        pltpu.sync_copy(
            out_sh.at[pl.ds(sid * RPT, RPT)],
            agg_hbm.at[cid].at[pl.ds(sid * RPT, RPT)])

    return k(src_all, dst_all, ex, den128, h_lo, h_hi)


def _dot(a, b):
    return jnp.dot(a, b, preferred_element_type=jnp.float32)


def _prep1_body(x_ref, W_ref, As_ref, Ad_ref, hlo_ref, hhi_ref, as_ref, ad_ref):
    h = _dot(x_ref[...], W_ref[...])
    hlo_ref[...] = h[:, :128]
    hhi_ref[...] = h[:, 128:]
    as_ref[...] = _dot(h, As_ref[...])
    ad_ref[...] = _dot(h, Ad_ref[...])


def _prep1(x_pad, W_pad, As, Ad):
    R = 2000
    grid = (N // R,)
    full = lambda a: pl.BlockSpec(a.shape, lambda i: (0,) * a.ndim)
    return pl.pallas_call(
        _prep1_body,
        grid=grid,
        in_specs=[pl.BlockSpec((R, 16), lambda i: (i, 0)),
                  full(W_pad), full(As), full(Ad)],
        out_specs=[pl.BlockSpec((R, 128), lambda i: (i, 0)),
                   pl.BlockSpec((R, 128), lambda i: (i, 0)),
                   pl.BlockSpec((R, 16), lambda i: (i, 0)),
                   pl.BlockSpec((R, 16), lambda i: (i, 0))],
        out_shape=[jax.ShapeDtypeStruct((N, 128), jnp.float32),
                   jax.ShapeDtypeStruct((N, 128), jnp.float32),
                   jax.ShapeDtypeStruct((N, 16), jnp.float32),
                   jax.ShapeDtypeStruct((N, 16), jnp.float32)],
    )(x_pad, W_pad, As, Ad)


def _prep_body(aglo_ref, aghi_ref, blo_ref, bhi_ref, Wt_ref, Wb_ref,
               As_ref, Ad_ref, hlo_ref, hhi_ref, as_ref, ad_ref):
    x_lo = jax.nn.relu(aglo_ref[...] + blo_ref[...])
    x_hi = jax.nn.relu(aghi_ref[...] + bhi_ref[...])
    h = _dot(x_lo, Wt_ref[...]) + _dot(x_hi, Wb_ref[...])
    D = h.shape[1]
    hlo_ref[...] = h[:, :D // 2]
    hhi_ref[...] = h[:, D // 2:]
    as_ref[...] = _dot(h, As_ref[...])
    ad_ref[...] = _dot(h, Ad_ref[...])


def _prep(agg_lo, agg_hi, b_lo, b_hi, W_top, W_bot, As, Ad, R=2000):
    D = W_top.shape[1]
    grid = (N // R,)
    full = lambda a: pl.BlockSpec(a.shape, lambda i: (0,) * a.ndim)
    return pl.pallas_call(
        _prep_body,
        grid=grid,
        in_specs=[pl.BlockSpec((R, 128), lambda i: (i, 0)),
                  pl.BlockSpec((R, 128), lambda i: (i, 0)),
                  full(b_lo), full(b_hi), full(W_top), full(W_bot),
                  full(As), full(Ad)],
        out_specs=[pl.BlockSpec((R, D // 2), lambda i: (i, 0)),
                   pl.BlockSpec((R, D // 2), lambda i: (i, 0)),
                   pl.BlockSpec((R, 16), lambda i: (i, 0)),
                   pl.BlockSpec((R, 16), lambda i: (i, 0))],
        out_shape=[jax.ShapeDtypeStruct((N, D // 2), jnp.float32),
                   jax.ShapeDtypeStruct((N, D // 2), jnp.float32),
                   jax.ShapeDtypeStruct((N, 16), jnp.float32),
                   jax.ShapeDtypeStruct((N, 16), jnp.float32)],
    )(agg_lo, agg_hi, b_lo, b_hi, W_top, W_bot, As, Ad)


def _cpack_body(as_ref, ad_ref, adc_ref):
    gmax = jnp.max(as_ref[...], axis=0, keepdims=True)
    z = gmax + ad_ref[...]
    c = jnp.where(z >= 0.0, z, 0.2 * z)
    adc_ref[...] = jnp.concatenate([ad_ref[...], c], axis=1)


def _cpack(asrc, adst):
    return pl.pallas_call(
        _cpack_body,
        out_shape=jax.ShapeDtypeStruct((N, 32), jnp.float32),
    )(asrc, adst)


def _tail_body(p0_ref, p1_ref, g4b_ref, fc_W_ref, fc_b_ref, out_W_ref,
               out_b_ref, x_trunk_ref, t1_W_ref, t1_b_ref, t2_W_ref,
               t2_b_ref, t3_W_ref, t3_b_ref, t4_W_ref, t4_b_ref, bias_ref,
               o_ref):
    x4 = (p0_ref[...] + p1_ref[...]) * jnp.float32(1.0 / 16.0) + g4b_ref[...]
    x5 = _dot(x4, fc_W_ref[...]) + fc_b_ref[...]
    bout = _dot(x5, out_W_ref[...]) + out_b_ref[...]
    t = jax.nn.relu(_dot(x_trunk_ref[...], t1_W_ref[...]) + t1_b_ref[...])
    t = jax.nn.relu(_dot(t, t2_W_ref[...]) + t2_b_ref[...])
    t = jax.nn.relu(_dot(t, t3_W_ref[...]) + t3_b_ref[...])
    t = jax.nn.relu(_dot(t, t4_W_ref[...]) + t4_b_ref[...])
    o_ref[...] = _dot(bout, t.T) + bias_ref[0]


def _tail(p0, p1, g4_b, fc_W, fc_b, out_W, out_b, x_trunk,
          t1_W, t1_b, t2_W, t2_b, t3_W, t3_b, t4_W, t4_b, bias):
    T = x_trunk.shape[0]
    R = 2000
    grid = (N // R,)
    full = lambda a: pl.BlockSpec(a.shape, lambda i: (0,) * a.ndim)
    return pl.pallas_call(
        _tail_body,
        grid=grid,
        in_specs=[pl.BlockSpec((R, 128), lambda i: (i, 0)),
                  pl.BlockSpec((R, 128), lambda i: (i, 0)),
                  full(g4_b), full(fc_W), full(fc_b), full(out_W),
                  full(out_b), full(x_trunk), full(t1_W), full(t1_b),
                  full(t2_W), full(t2_b), full(t3_W), full(t3_b),
                  full(t4_W), full(t4_b), full(bias)],
        out_specs=pl.BlockSpec((R, T), lambda i: (i, 0)),
        out_shape=jax.ShapeDtypeStruct((N, T), jnp.float32),
    )(p0, p1, g4_b, fc_W, fc_b, out_W, out_b, x_trunk,
      t1_W, t1_b, t2_W, t2_b, t3_W, t3_b, t4_W, t4_b, bias)


def _expand_attn(a):
    H, C = a.shape
    return (a[:, :, None] * jnp.eye(H, dtype=a.dtype)[:, None, :]).reshape(
        H * C, H)


def _pad_adc(adc):
    pad = jnp.concatenate(
        [jnp.zeros((N_ACC - N, 16), jnp.float32),
         jnp.full((N_ACC - N, 16), 1e30, jnp.float32)], axis=1)
    return jnp.pad(jnp.concatenate([adc, pad]), ((0, 0), (0, 96)))


def _pad_lanes(a):
    return jnp.pad(a, ((0, 0), (0, 128 - a.shape[1])))


def _den128(den):
    return jnp.pad(jnp.concatenate([den[0], den[1]], axis=1),
                   ((0, 0), (0, 96)))


def kernel(x_branch, x_branch_index, x_trunk, g1_W, g1_as, g1_ad, g1_b,
           g2_W, g2_as, g2_ad, g2_b, g3_W, g3_as, g3_ad, g3_b,
           g4_W, g4_as, g4_ad, g4_b, fc_W, fc_b, out_W, out_b,
           t1_W, t1_b, t2_W, t2_b, t3_W, t3_b, t4_W, t4_b, bias):
    src = x_branch_index[0].astype(jnp.int32)
    dst = x_branch_index[1].astype(jnp.int32)
    dst_m = jnp.where(src == dst, TRASH, dst)
    loop = jnp.arange(N, dtype=jnp.int32)
    n_pad = E_PAD - EN
    src_all = jnp.concatenate([src, loop, jnp.zeros((n_pad,), jnp.int32)])
    dst_all = jnp.concatenate(
        [dst_m, loop, jnp.full((n_pad,), TRASH, jnp.int32)])

    x_pad = jnp.pad(x_branch, ((0, 0), (0, 3)))
    W1_pad = jnp.pad(g1_W, ((0, 3), (0, 0)))

    h_lo, h_hi, asrc, adst = _prep1(x_pad, W1_pad, _expand_attn(g1_as),
                                    _expand_attn(g1_ad))
    adc = _pad_adc(_cpack(asrc, adst))
    ex, den = _attn_pass(src_all, dst_all, _pad_lanes(asrc), adc)
    agg = _agg_pass(src_all, dst_all, ex, _den128(den), h_lo, h_hi)
    h_lo, h_hi, asrc, adst = _prep(agg[0][:N], agg[1][:N],
                                   g1_b[:128], g1_b[128:],
                                   g2_W[:128], g2_W[128:],
                                   _expand_attn(g2_as), _expand_attn(g2_ad))
    adc = _pad_adc(_cpack(asrc, adst))
    ex, den = _attn_pass(src_all, dst_all, _pad_lanes(asrc), adc)
    agg = _agg_pass(src_all, dst_all, ex, _den128(den), h_lo, h_hi)
    h_lo, h_hi, asrc, adst = _prep(agg[0][:N], agg[1][:N],
                                   g2_b[:128], g2_b[128:],
                                   g3_W[:128], g3_W[128:],
                                   _expand_attn(g3_as), _expand_attn(g3_ad))
    adc = _pad_adc(_cpack(asrc, adst))
    ex, den = _attn_pass(src_all, dst_all, _pad_lanes(asrc), adc)
    agg = _agg_pass(src_all, dst_all, ex, _den128(den), h_lo, h_hi)
    h_lo, h_hi, asrc, adst = _prep(agg[0][:N], agg[1][:N],
                                   g3_b[:128], g3_b[128:],
                                   g4_W[:128], g4_W[128:],
                                   _expand_attn(g4_as), _expand_attn(g4_ad),
                                   R=1000)
    adc = _pad_adc(_cpack(asrc, adst))
    ex, den = _attn_pass(src_all, dst_all, _pad_lanes(asrc), adc)
    p = _agg4_pass(src_all, dst_all, ex, _den128(den), h_lo, h_hi)
    return _tail(p[0][:N], p[1][:N], g4_b, fc_W, fc_b, out_W, out_b, x_trunk,
                 t1_W, t1_b, t2_W, t2_b, t3_W, t3_b, t4_W, t4_b, bias)


# speedup vs baseline: 20.1001x; 1.1757x over previous
import functools

import jax
import jax.numpy as jnp
from jax import lax
from jax.experimental import pallas as pl
from jax.experimental.pallas import tpu as pltpu
from jax.experimental.pallas import tpu_sc as plsc

N = 10000
E = 320000
EN = E + N
E_PAD = 330240
TRASH = N
N_ACC = N + 16
ROWS = 10112
RPT = ROWS // 16
K1 = 80
K4 = 24


def _mesh():
    return plsc.VectorSubcoreMesh(core_axis_name="c", subcore_axis_name="s")


def _zero_rows(z_buf, shared, row0, nrows):
    D = shared.shape[1]
    rpc = z_buf.shape[0]

    def zb(r, _):
        for j in range(D // 16):
            z_buf[r, pl.ds(j * 16, 16)] = jnp.zeros((16,), jnp.float32)
        return 0

    lax.fori_loop(0, rpc, zb, 0)
    for j in range(nrows // rpc):
        pltpu.sync_copy(z_buf, shared.at[pl.ds(row0 + j * rpc, rpc)])


def _attn_pass(src_all, dst_all, asrc, adc):
    mesh = _mesh()
    n_chunks = (E_PAD // 2) // 16 // K1

    @functools.partial(
        pl.kernel, mesh=mesh,
        out_type=(jax.ShapeDtypeStruct((E_PAD, 16), jnp.float32),
                  jax.ShapeDtypeStruct((2, ROWS, 16), jnp.float32)),
        scratch_types=[
            pltpu.VMEM((K1,), jnp.int32),
            pltpu.VMEM((K1,), jnp.int32),
            pltpu.VMEM((K1, 128), jnp.float32),
            pltpu.VMEM((K1, 128), jnp.float32),
            pltpu.VMEM((K1, 16), jnp.float32),
            pltpu.VMEM((8, 16), jnp.float32),
            pltpu.VMEM_SHARED((ROWS, 16), jnp.float32),
            pltpu.SemaphoreType.DMA,
        ],
    )
    def k(src_hbm, dst_hbm, asrc_hbm, adc_hbm, ex_hbm, den_hbm,
          idx_s, idx_d, as_buf, adc_buf, ex_buf, zd_buf, den_sh, sem):
        cid = lax.axis_index("c")
        sid = lax.axis_index("s")
        _zero_rows(zd_buf, den_sh, sid * RPT, RPT)
        plsc.subcore_barrier()
        tile_base = cid * (E_PAD // 2) + sid * (E_PAD // 32)

        def edge(e, _):
            z = as_buf[e, pl.ds(0, 16)] + adc_buf[e, pl.ds(0, 16)]
            al = jnp.where(z >= 0.0, z, 0.2 * z)
            ex_buf[e] = jnp.exp(al - adc_buf[e, pl.ds(16, 16)])
            return 0

        def chunk(ch, _):
            base = tile_base + ch * K1
            pltpu.sync_copy(src_hbm.at[pl.ds(base, K1)], idx_s)
            pltpu.sync_copy(dst_hbm.at[pl.ds(base, K1)], idx_d)
            ga = pltpu.async_copy(asrc_hbm.at[idx_s], as_buf, sem)
            gd = pltpu.async_copy(adc_hbm.at[idx_d], adc_buf, sem)
            ga.wait()
            gd.wait()
            lax.fori_loop(0, K1, edge, 0)
            pltpu.sync_copy(ex_buf, ex_hbm.at[pl.ds(base, K1)])
            pltpu.sync_copy(ex_buf, den_sh.at[idx_d], add=True)
            return 0

        lax.fori_loop(0, n_chunks, chunk, 0)
        plsc.subcore_barrier()
        pltpu.sync_copy(
            den_sh.at[pl.ds(sid * RPT, RPT)],
            den_hbm.at[cid].at[pl.ds(sid * RPT, RPT)])

    return k(src_all, dst_all, asrc, adc)


def _agg_pass(src_all, dst_all, ex, den128, h_lo, h_hi):
    mesh = _mesh()
    n_chunks = E_PAD // 16 // K1

    @functools.partial(
        pl.kernel, mesh=mesh,
        out_type=jax.ShapeDtypeStruct((2, ROWS, 128), jnp.float32),
        scratch_types=[
            pltpu.VMEM((K1,), jnp.int32),
            pltpu.VMEM((K1,), jnp.int32),
            pltpu.VMEM((K1, 16), jnp.float32),
            pltpu.VMEM((K1, 128), jnp.float32),
            pltpu.VMEM((K1, 128), jnp.float32),
            pltpu.VMEM((K1, 128), jnp.float32),
            pltpu.VMEM((8, 128), jnp.float32),
            pltpu.VMEM_SHARED((ROWS, 128), jnp.float32),
            pltpu.SemaphoreType.DMA,
        ],
    )
    def k(src_hbm, dst_hbm, ex_hbm, den_hbm, hlo_hbm, hhi_hbm,
          agg_hbm, idx_s, idx_d, ex_buf, d_buf, h_buf, ob_buf, zo_buf,
          out_sh, sem):
        cid = lax.axis_index("c")
        sid = lax.axis_index("s")
        _zero_rows(zo_buf, out_sh, sid * RPT, RPT)
        plsc.subcore_barrier()
        tile_base = sid * (E_PAD // 16)

        def make_edge(ho):
            def edge(e, _):
                den = (d_buf[e, pl.ds(0, 16)] + d_buf[e, pl.ds(16, 16)]
                       + jnp.float32(1e-16))
                coef = ex_buf[e] / den
                for h in range(8):
                    bc = lax.broadcast(coef[ho + h], (16,))
                    ob_buf[e, pl.ds(h * 16, 16)] = (
                        bc * h_buf[e, pl.ds(h * 16, 16)])
                return 0
            return edge

        def chunk(ch, _):
            base = tile_base + ch * K1
            pltpu.sync_copy(src_hbm.at[pl.ds(base, K1)], idx_s)
            pltpu.sync_copy(dst_hbm.at[pl.ds(base, K1)], idx_d)
            pltpu.sync_copy(ex_hbm.at[pl.ds(base, K1)], ex_buf)
            g0 = pltpu.async_copy(den_hbm.at[idx_d], d_buf, sem)

            @pl.when(cid == 0)
            def _():
                pltpu.async_copy(hlo_hbm.at[idx_s], h_buf, sem).wait()

            @pl.when(cid != 0)
            def _():
                pltpu.async_copy(hhi_hbm.at[idx_s], h_buf, sem).wait()

            g0.wait()

            @pl.when(cid == 0)
            def _():
                lax.fori_loop(0, K1, make_edge(0), 0)

            @pl.when(cid != 0)
            def _():
                lax.fori_loop(0, K1, make_edge(8), 0)

            pltpu.sync_copy(ob_buf, out_sh.at[idx_d], add=True)
            return 0

        lax.fori_loop(0, n_chunks, chunk, 0)
        plsc.subcore_barrier()
        pltpu.sync_copy(
            out_sh.at[pl.ds(sid * RPT, RPT)],
            agg_hbm.at[cid].at[pl.ds(sid * RPT, RPT)])

    return k(src_all, dst_all, ex, den128, h_lo, h_hi)


def _agg4_pass(src_all, dst_all, ex, den128, h_lo, h_hi):
    mesh = _mesh()
    n_chunks = E_PAD // 16 // K4

    @functools.partial(
        pl.kernel, mesh=mesh,
        out_type=jax.ShapeDtypeStruct((2, ROWS, 128), jnp.float32),
        scratch_types=[
            pltpu.VMEM((K4,), jnp.int32),
            pltpu.VMEM((K4,), jnp.int32),
            pltpu.VMEM((K4, 16), jnp.float32),
            pltpu.VMEM((K4, 128), jnp.float32),
            pltpu.VMEM((K4, 1024), jnp.float32),
            pltpu.VMEM((K4, 128), jnp.float32),
            pltpu.VMEM((8, 128), jnp.float32),
            pltpu.VMEM_SHARED((ROWS, 128), jnp.float32),
            pltpu.SemaphoreType.DMA,
        ],
    )
    def k(src_hbm, dst_hbm, ex_hbm, den_hbm, hlo_hbm, hhi_hbm,
          agg_hbm, idx_s, idx_d, ex_buf, d_buf, h_buf,
          ob_buf, zo_buf, out_sh, sem):
        cid = lax.axis_index("c")
        sid = lax.axis_index("s")
        _zero_rows(zo_buf, out_sh, sid * RPT, RPT)
        plsc.subcore_barrier()
        tile_base = sid * (E_PAD // 16)

        def make_edge(ho):
            def edge(e, _):
                den = (d_buf[e, pl.ds(0, 16)] + d_buf[e, pl.ds(16, 16)]
                       + jnp.float32(1e-16))
                coef = ex_buf[e] / den
                acc = [jnp.zeros((16,), jnp.float32) for _ in range(8)]
                for h in range(8):
                    bc = lax.broadcast(coef[ho + h], (16,))
                    for j in range(8):
                        acc[j] = acc[j] + bc * h_buf[
                            e, pl.ds(h * 128 + j * 16, 16)]
                for j in range(8):
                    ob_buf[e, pl.ds(j * 16, 16)] = acc[j]
                return 0
            return edge

        def chunk(ch, _):
            base = tile_base + ch * K4
            pltpu.sync_copy(src_hbm.at[pl.ds(base, K4)], idx_s)
            pltpu.sync_copy(dst_hbm.at[pl.ds(base, K4)], idx_d)
            pltpu.sync_copy(ex_hbm.at[pl.ds(base, K4)], ex_buf)
            g0 = pltpu.async_copy(den_hbm.at[idx_d], d_buf, sem)

            @pl.when(cid == 0)
            def _():
                pltpu.async_copy(hlo_hbm.at[idx_s], h_buf, sem).wait()

            @pl.when(cid != 0)
            def _():
                pltpu.async_copy(hhi_hbm.at[idx_s], h_buf, sem).wait()

            g0.wait()

            @pl.when(cid == 0)
            def _():
                lax.fori_loop(0, K4, make_edge(0), 0)

            @pl.when(cid != 0)
            def _():
                lax.fori_loop(0, K4, make_edge(8), 0)

            pltpu.sync_copy(ob_buf, out_sh.at[idx_d], add=True)
            return 0

        lax.fori_loop(0, n_chunks, chunk, 0)
        plsc.subcore_barrier()
        pltpu.sync_copy(
            out_sh.at[pl.ds(sid * RPT, RPT)],
            agg_hbm.at[cid].at[pl.ds(sid * RPT, RPT)])

    return k(src_all, dst_all, ex, den128, h_lo, h_hi)


def _dot(a, b):
    return jnp.dot(a, b, preferred_element_type=jnp.float32)


def _prep1_body(x_ref, W_ref, As_ref, Ad_ref, hlo_ref, hhi_ref, as_ref, ad_ref):
    h = _dot(x_ref[...], W_ref[...])
    hlo_ref[...] = h[:, :128]
    hhi_ref[...] = h[:, 128:]
    as_ref[...] = _dot(h, As_ref[...])
    ad_ref[...] = _dot(h, Ad_ref[...])


def _prep1(x_pad, W_pad, As, Ad):
    R = 2000
    grid = (N // R,)
    full = lambda a: pl.BlockSpec(a.shape, lambda i: (0,) * a.ndim)
    return pl.pallas_call(
        _prep1_body,
        grid=grid,
        in_specs=[pl.BlockSpec((R, 16), lambda i: (i, 0)),
                  full(W_pad), full(As), full(Ad)],
        out_specs=[pl.BlockSpec((R, 128), lambda i: (i, 0)),
                   pl.BlockSpec((R, 128), lambda i: (i, 0)),
                   pl.BlockSpec((R, 16), lambda i: (i, 0)),
                   pl.BlockSpec((R, 16), lambda i: (i, 0))],
        out_shape=[jax.ShapeDtypeStruct((N, 128), jnp.float32),
                   jax.ShapeDtypeStruct((N, 128), jnp.float32),
                   jax.ShapeDtypeStruct((N, 16), jnp.float32),
                   jax.ShapeDtypeStruct((N, 16), jnp.float32)],
    )(x_pad, W_pad, As, Ad)


def _prep_body(aglo_ref, aghi_ref, blo_ref, bhi_ref, Wt_ref, Wb_ref,
               As_ref, Ad_ref, hlo_ref, hhi_ref, as_ref, ad_ref):
    x_lo = jax.nn.relu(aglo_ref[...] + blo_ref[...])
    x_hi = jax.nn.relu(aghi_ref[...] + bhi_ref[...])
    h = _dot(x_lo, Wt_ref[...]) + _dot(x_hi, Wb_ref[...])
    D = h.shape[1]
    hlo_ref[...] = h[:, :D // 2]
    hhi_ref[...] = h[:, D // 2:]
    as_ref[...] = _dot(h, As_ref[...])
    ad_ref[...] = _dot(h, Ad_ref[...])


def _prep(agg_lo, agg_hi, b_lo, b_hi, W_top, W_bot, As, Ad, R=2000):
    D = W_top.shape[1]
    grid = (N // R,)
    full = lambda a: pl.BlockSpec(a.shape, lambda i: (0,) * a.ndim)
    return pl.pallas_call(
        _prep_body,
        grid=grid,
        in_specs=[pl.BlockSpec((R, 128), lambda i: (i, 0)),
                  pl.BlockSpec((R, 128), lambda i: (i, 0)),
                  full(b_lo), full(b_hi), full(W_top), full(W_bot),
                  full(As), full(Ad)],
        out_specs=[pl.BlockSpec((R, D // 2), lambda i: (i, 0)),
                   pl.BlockSpec((R, D // 2), lambda i: (i, 0)),
                   pl.BlockSpec((R, 16), lambda i: (i, 0)),
                   pl.BlockSpec((R, 16), lambda i: (i, 0))],
        out_shape=[jax.ShapeDtypeStruct((N, D // 2), jnp.float32),
                   jax.ShapeDtypeStruct((N, D // 2), jnp.float32),
                   jax.ShapeDtypeStruct((N, 16), jnp.float32),
                   jax.ShapeDtypeStruct((N, 16), jnp.float32)],
    )(agg_lo, agg_hi, b_lo, b_hi, W_top, W_bot, As, Ad)


def _cpack_body(as_ref, ad_ref, adc_ref):
    gmax = jnp.max(as_ref[...], axis=0, keepdims=True)
    z = gmax + ad_ref[...]
    c = jnp.where(z >= 0.0, z, 0.2 * z)
    adc_ref[...] = jnp.concatenate([ad_ref[...], c], axis=1)


def _cpack(asrc, adst):
    return pl.pallas_call(
        _cpack_body,
        out_shape=jax.ShapeDtypeStruct((N, 32), jnp.float32),
    )(asrc, adst)


def _tail_body(p0_ref, p1_ref, g4b_ref, fc_W_ref, fc_b_ref, out_W_ref,
               out_b_ref, x_trunk_ref, t1_W_ref, t1_b_ref, t2_W_ref,
               t2_b_ref, t3_W_ref, t3_b_ref, t4_W_ref, t4_b_ref, bias_ref,
               o_ref):
    x4 = (p0_ref[...] + p1_ref[...]) * jnp.float32(1.0 / 16.0) + g4b_ref[...]
    x5 = _dot(x4, fc_W_ref[...]) + fc_b_ref[...]
    bout = _dot(x5, out_W_ref[...]) + out_b_ref[...]
    t = jax.nn.relu(_dot(x_trunk_ref[...], t1_W_ref[...]) + t1_b_ref[...])
    t = jax.nn.relu(_dot(t, t2_W_ref[...]) + t2_b_ref[...])
    t = jax.nn.relu(_dot(t, t3_W_ref[...]) + t3_b_ref[...])
    t = jax.nn.relu(_dot(t, t4_W_ref[...]) + t4_b_ref[...])
    o_ref[...] = _dot(bout, t.T) + bias_ref[0]


def _tail(p0, p1, g4_b, fc_W, fc_b, out_W, out_b, x_trunk,
          t1_W, t1_b, t2_W, t2_b, t3_W, t3_b, t4_W, t4_b, bias):
    T = x_trunk.shape[0]
    R = 2000
    grid = (N // R,)
    full = lambda a: pl.BlockSpec(a.shape, lambda i: (0,) * a.ndim)
    return pl.pallas_call(
        _tail_body,
        grid=grid,
        in_specs=[pl.BlockSpec((R, 128), lambda i: (i, 0)),
                  pl.BlockSpec((R, 128), lambda i: (i, 0)),
                  full(g4_b), full(fc_W), full(fc_b), full(out_W),
                  full(out_b), full(x_trunk), full(t1_W), full(t1_b),
                  full(t2_W), full(t2_b), full(t3_W), full(t3_b),
                  full(t4_W), full(t4_b), full(bias)],
        out_specs=pl.BlockSpec((R, T), lambda i: (i, 0)),
        out_shape=jax.ShapeDtypeStruct((N, T), jnp.float32),
    )(p0, p1, g4_b, fc_W, fc_b, out_W, out_b, x_trunk,
      t1_W, t1_b, t2_W, t2_b, t3_W, t3_b, t4_W, t4_b, bias)


def _expand_attn(a):
    H, C = a.shape
    return (a[:, :, None] * jnp.eye(H, dtype=a.dtype)[:, None, :]).reshape(
        H * C, H)


def _pad_adc(adc):
    pad = jnp.concatenate(
        [jnp.zeros((N_ACC - N, 16), jnp.float32),
         jnp.full((N_ACC - N, 16), 1e30, jnp.float32)], axis=1)
    return jnp.pad(jnp.concatenate([adc, pad]), ((0, 0), (0, 96)))


def _pad_lanes(a):
    return jnp.pad(a, ((0, 0), (0, 128 - a.shape[1])))


def _den128(den):
    return jnp.pad(jnp.concatenate([den[0], den[1]], axis=1),
                   ((0, 0), (0, 96)))


def kernel(x_branch, x_branch_index, x_trunk, g1_W, g1_as, g1_ad, g1_b,
           g2_W, g2_as, g2_ad, g2_b, g3_W, g3_as, g3_ad, g3_b,
           g4_W, g4_as, g4_ad, g4_b, fc_W, fc_b, out_W, out_b,
           t1_W, t1_b, t2_W, t2_b, t3_W, t3_b, t4_W, t4_b, bias):
    src = x_branch_index[0].astype(jnp.int32)
    dst = x_branch_index[1].astype(jnp.int32)
    dst_m = jnp.where(src == dst, TRASH, dst)
    loop = jnp.arange(N, dtype=jnp.int32)
    n_pad = E_PAD - EN
    src_all = jnp.concatenate([src, loop, jnp.zeros((n_pad,), jnp.int32)])
    dst_all = jnp.concatenate(
        [dst_m, loop, jnp.full((n_pad,), TRASH, jnp.int32)])

    x_pad = jnp.pad(x_branch, ((0, 0), (0, 3)))
    W1_pad = jnp.pad(g1_W, ((0, 3), (0, 0)))

    h_lo, h_hi, asrc, adst = _prep1(x_pad, W1_pad, _expand_attn(g1_as),
                                    _expand_attn(g1_ad))
    adc = _pad_adc(_cpack(asrc, adst))
    ex, den = _attn_pass(src_all, dst_all, _pad_lanes(asrc), adc)
    agg = _agg_pass(src_all, dst_all, ex, _den128(den), h_lo, h_hi)
    h_lo, h_hi, asrc, adst = _prep(agg[0][:N], agg[1][:N],
                                   g1_b[:128], g1_b[128:],
                                   g2_W[:128], g2_W[128:],
                                   _expand_attn(g2_as), _expand_attn(g2_ad))
    adc = _pad_adc(_cpack(asrc, adst))
    ex, den = _attn_pass(src_all, dst_all, _pad_lanes(asrc), adc)
    agg = _agg_pass(src_all, dst_all, ex, _den128(den), h_lo, h_hi)
    h_lo, h_hi, asrc, adst = _prep(agg[0][:N], agg[1][:N],
                                   g2_b[:128], g2_b[128:],
                                   g3_W[:128], g3_W[128:],
                                   _expand_attn(g3_as), _expand_attn(g3_ad))
    adc = _pad_adc(_cpack(asrc, adst))
    ex, den = _attn_pass(src_all, dst_all, _pad_lanes(asrc), adc)
    agg = _agg_pass(src_all, dst_all, ex, _den128(den), h_lo, h_hi)
    h_lo, h_hi, asrc, adst = _prep(agg[0][:N], agg[1][:N],
                                   g3_b[:128], g3_b[128:],
                                   g4_W[:128], g4_W[128:],
                                   _expand_attn(g4_as), _expand_attn(g4_ad),
                                   R=1000)
    adc = _pad_adc(_cpack(asrc, adst))
    ex, den = _attn_pass(src_all, dst_all, _pad_lanes(asrc), adc)
    p = _agg4_pass(src_all, dst_all, ex, _den128(den), h_lo, h_hi)
    return _tail(p[0][:N], p[1][:N], g4_b, fc_W, fc_b, out_W, out_b, x_trunk,
                 t1_W, t1_b, t2_W, t2_b, t3_W, t3_b, t4_W, t4_b, bias)


# speedup vs baseline: 21.0594x; 1.0477x over previous
import functools

import jax
import jax.numpy as jnp
from jax import lax
from jax.experimental import pallas as pl
from jax.experimental.pallas import tpu as pltpu
from jax.experimental.pallas import tpu_sc as plsc

N = 10000
E = 320000
EN = E + N
E_PAD = 330240
TRASH = N
N_ACC = N + 16
ROWS = 10112
RPT = ROWS // 16
K1 = 80
K4 = 32


def _mesh():
    return plsc.VectorSubcoreMesh(core_axis_name="c", subcore_axis_name="s")


def _zero_rows(z_buf, shared, row0, nrows):
    D = shared.shape[1]
    rpc = z_buf.shape[0]

    def zb(r, _):
        for j in range(D // 16):
            z_buf[r, pl.ds(j * 16, 16)] = jnp.zeros((16,), jnp.float32)
        return 0

    lax.fori_loop(0, rpc, zb, 0)
    for j in range(nrows // rpc):
        pltpu.sync_copy(z_buf, shared.at[pl.ds(row0 + j * rpc, rpc)])


def _attn_pass(src_all, dst_all, asrc, adc):
    mesh = _mesh()
    n_chunks = (E_PAD // 2) // 16 // K1

    @functools.partial(
        pl.kernel, mesh=mesh,
        out_type=(jax.ShapeDtypeStruct((E_PAD, 16), jnp.float32),
                  jax.ShapeDtypeStruct((2, ROWS, 16), jnp.float32)),
        scratch_types=[
            pltpu.VMEM((K1,), jnp.int32),
            pltpu.VMEM((K1,), jnp.int32),
            pltpu.VMEM((K1, 128), jnp.float32),
            pltpu.VMEM((K1, 128), jnp.float32),
            pltpu.VMEM((K1, 16), jnp.float32),
            pltpu.VMEM((8, 16), jnp.float32),
            pltpu.VMEM_SHARED((ROWS, 16), jnp.float32),
            pltpu.SemaphoreType.DMA,
        ],
    )
    def k(src_hbm, dst_hbm, asrc_hbm, adc_hbm, ex_hbm, den_hbm,
          idx_s, idx_d, as_buf, adc_buf, ex_buf, zd_buf, den_sh, sem):
        cid = lax.axis_index("c")
        sid = lax.axis_index("s")
        _zero_rows(zd_buf, den_sh, sid * RPT, RPT)
        plsc.subcore_barrier()
        tile_base = cid * (E_PAD // 2) + sid * (E_PAD // 32)

        def edge(e, _):
            z = as_buf[e, pl.ds(0, 16)] + adc_buf[e, pl.ds(0, 16)]
            al = jnp.where(z >= 0.0, z, 0.2 * z)
            ex_buf[e] = jnp.exp(al - adc_buf[e, pl.ds(16, 16)])
            return 0

        def chunk(ch, _):
            base = tile_base + ch * K1
            pltpu.sync_copy(src_hbm.at[pl.ds(base, K1)], idx_s)
            pltpu.sync_copy(dst_hbm.at[pl.ds(base, K1)], idx_d)
            ga = pltpu.async_copy(asrc_hbm.at[idx_s], as_buf, sem)
            gd = pltpu.async_copy(adc_hbm.at[idx_d], adc_buf, sem)
            ga.wait()
            gd.wait()
            lax.fori_loop(0, K1, edge, 0)
            pltpu.sync_copy(ex_buf, ex_hbm.at[pl.ds(base, K1)])
            pltpu.sync_copy(ex_buf, den_sh.at[idx_d], add=True)
            return 0

        lax.fori_loop(0, n_chunks, chunk, 0)
        plsc.subcore_barrier()
        pltpu.sync_copy(
            den_sh.at[pl.ds(sid * RPT, RPT)],
            den_hbm.at[cid].at[pl.ds(sid * RPT, RPT)])

    return k(src_all, dst_all, asrc, adc)


def _agg_pass(src_all, dst_all, ex, den128, h_lo, h_hi):
    mesh = _mesh()
    n_chunks = E_PAD // 16 // K1

    @functools.partial(
        pl.kernel, mesh=mesh,
        out_type=jax.ShapeDtypeStruct((2, ROWS, 128), jnp.float32),
        scratch_types=[
            pltpu.VMEM((K1,), jnp.int32),
            pltpu.VMEM((K1,), jnp.int32),
            pltpu.VMEM((K1, 16), jnp.float32),
            pltpu.VMEM((K1, 128), jnp.float32),
            pltpu.VMEM((K1, 128), jnp.float32),
            pltpu.VMEM((K1, 128), jnp.float32),
            pltpu.VMEM((8, 128), jnp.float32),
            pltpu.VMEM_SHARED((ROWS, 128), jnp.float32),
            pltpu.SemaphoreType.DMA,
        ],
    )
    def k(src_hbm, dst_hbm, ex_hbm, den_hbm, hlo_hbm, hhi_hbm,
          agg_hbm, idx_s, idx_d, ex_buf, d_buf, h_buf, ob_buf, zo_buf,
          out_sh, sem):
        cid = lax.axis_index("c")
        sid = lax.axis_index("s")
        _zero_rows(zo_buf, out_sh, sid * RPT, RPT)
        plsc.subcore_barrier()
        tile_base = sid * (E_PAD // 16)

        def make_edge(ho):
            def edge(e, _):
                den = (d_buf[e, pl.ds(0, 16)] + d_buf[e, pl.ds(16, 16)]
                       + jnp.float32(1e-16))
                coef = ex_buf[e] / den
                for h in range(8):
                    bc = lax.broadcast(coef[ho + h], (16,))
                    ob_buf[e, pl.ds(h * 16, 16)] = (
                        bc * h_buf[e, pl.ds(h * 16, 16)])
                return 0
            return edge

        def chunk(ch, _):
            base = tile_base + ch * K1
            pltpu.sync_copy(src_hbm.at[pl.ds(base, K1)], idx_s)
            pltpu.sync_copy(dst_hbm.at[pl.ds(base, K1)], idx_d)
            pltpu.sync_copy(ex_hbm.at[pl.ds(base, K1)], ex_buf)
            g0 = pltpu.async_copy(den_hbm.at[idx_d], d_buf, sem)

            @pl.when(cid == 0)
            def _():
                pltpu.async_copy(hlo_hbm.at[idx_s], h_buf, sem).wait()

            @pl.when(cid != 0)
            def _():
                pltpu.async_copy(hhi_hbm.at[idx_s], h_buf, sem).wait()

            g0.wait()

            @pl.when(cid == 0)
            def _():
                lax.fori_loop(0, K1, make_edge(0), 0)

            @pl.when(cid != 0)
            def _():
                lax.fori_loop(0, K1, make_edge(8), 0)

            pltpu.sync_copy(ob_buf, out_sh.at[idx_d], add=True)
            return 0

        lax.fori_loop(0, n_chunks, chunk, 0)
        plsc.subcore_barrier()
        pltpu.sync_copy(
            out_sh.at[pl.ds(sid * RPT, RPT)],
            agg_hbm.at[cid].at[pl.ds(sid * RPT, RPT)])

    return k(src_all, dst_all, ex, den128, h_lo, h_hi)


def _agg4_pass(src_all, dst_all, ex, den128, h_lo, h_hi):
    mesh = _mesh()
    n_chunks = E_PAD // 16 // K4

    @functools.partial(
        pl.kernel, mesh=mesh,
        out_type=jax.ShapeDtypeStruct((2, ROWS, 128), jnp.float32),
        scratch_types=[
            pltpu.VMEM((K4,), jnp.int32),
            pltpu.VMEM((K4,), jnp.int32),
            pltpu.VMEM((K4, 16), jnp.float32),
            pltpu.VMEM((K4, 128), jnp.float32),
            pltpu.VMEM((K4, 1024), jnp.float32),
            pltpu.VMEM((K4, 128), jnp.float32),
            pltpu.VMEM((8, 128), jnp.float32),
            pltpu.VMEM_SHARED((ROWS, 128), jnp.float32),
            pltpu.SemaphoreType.DMA,
        ],
    )
    def k(src_hbm, dst_hbm, ex_hbm, den_hbm, hlo_hbm, hhi_hbm,
          agg_hbm, idx_s, idx_d, ex_buf, d_buf, h_buf,
          ob_buf, zo_buf, out_sh, sem):
        cid = lax.axis_index("c")
        sid = lax.axis_index("s")
        _zero_rows(zo_buf, out_sh, sid * RPT, RPT)
        plsc.subcore_barrier()
        tile_base = sid * (E_PAD // 16)

        def make_edge(ho):
            def edge(e, _):
                den = (d_buf[e, pl.ds(0, 16)] + d_buf[e, pl.ds(16, 16)]
                       + jnp.float32(1e-16))
                coef = ex_buf[e] / den
                acc = [jnp.zeros((16,), jnp.float32) for _ in range(8)]
                for h in range(8):
                    bc = lax.broadcast(coef[ho + h], (16,))
                    for j in range(8):
                        acc[j] = acc[j] + bc * h_buf[
                            e, pl.ds(h * 128 + j * 16, 16)]
                for j in range(8):
                    ob_buf[e, pl.ds(j * 16, 16)] = acc[j]
                return 0
            return edge

        def chunk(ch, _):
            base = tile_base + ch * K4
            pltpu.sync_copy(src_hbm.at[pl.ds(base, K4)], idx_s)
            pltpu.sync_copy(dst_hbm.at[pl.ds(base, K4)], idx_d)
            pltpu.sync_copy(ex_hbm.at[pl.ds(base, K4)], ex_buf)
            g0 = pltpu.async_copy(den_hbm.at[idx_d], d_buf, sem)

            @pl.when(cid == 0)
            def _():
                pltpu.async_copy(hlo_hbm.at[idx_s], h_buf, sem).wait()

            @pl.when(cid != 0)
            def _():
                pltpu.async_copy(hhi_hbm.at[idx_s], h_buf, sem).wait()

            g0.wait()

            @pl.when(cid == 0)
            def _():
                lax.fori_loop(0, K4, make_edge(0), 0)

            @pl.when(cid != 0)
            def _():
                lax.fori_loop(0, K4, make_edge(8), 0)

            pltpu.sync_copy(ob_buf, out_sh.at[idx_d], add=True)
            return 0

        lax.fori_loop(0, n_chunks, chunk, 0)
        plsc.subcore_barrier()
        pltpu.sync_copy(
            out_sh.at[pl.ds(sid * RPT, RPT)],
            agg_hbm.at[cid].at[pl.ds(sid * RPT, RPT)])

    return k(src_all, dst_all, ex, den128, h_lo, h_hi)


def _dot(a, b):
    return jnp.dot(a, b, preferred_element_type=jnp.float32)


def _prep1_body(x_ref, W_ref, As_ref, Ad_ref, hlo_ref, hhi_ref, as_ref, ad_ref):
    h = _dot(x_ref[...], W_ref[...])
    hlo_ref[...] = h[:, :128]
    hhi_ref[...] = h[:, 128:]
    as_ref[...] = _dot(h, As_ref[...])
    ad_ref[...] = _dot(h, Ad_ref[...])


def _prep1(x_pad, W_pad, As, Ad):
    R = 2000
    grid = (N // R,)
    full = lambda a: pl.BlockSpec(a.shape, lambda i: (0,) * a.ndim)
    return pl.pallas_call(
        _prep1_body,
        grid=grid,
        in_specs=[pl.BlockSpec((R, 16), lambda i: (i, 0)),
                  full(W_pad), full(As), full(Ad)],
        out_specs=[pl.BlockSpec((R, 128), lambda i: (i, 0)),
                   pl.BlockSpec((R, 128), lambda i: (i, 0)),
                   pl.BlockSpec((R, 16), lambda i: (i, 0)),
                   pl.BlockSpec((R, 16), lambda i: (i, 0))],
        out_shape=[jax.ShapeDtypeStruct((N, 128), jnp.float32),
                   jax.ShapeDtypeStruct((N, 128), jnp.float32),
                   jax.ShapeDtypeStruct((N, 16), jnp.float32),
                   jax.ShapeDtypeStruct((N, 16), jnp.float32)],
    )(x_pad, W_pad, As, Ad)


def _prep_body(aglo_ref, aghi_ref, blo_ref, bhi_ref, Wt_ref, Wb_ref,
               As_ref, Ad_ref, hlo_ref, hhi_ref, as_ref, ad_ref):
    x_lo = jax.nn.relu(aglo_ref[...] + blo_ref[...])
    x_hi = jax.nn.relu(aghi_ref[...] + bhi_ref[...])
    h = _dot(x_lo, Wt_ref[...]) + _dot(x_hi, Wb_ref[...])
    D = h.shape[1]
    hlo_ref[...] = h[:, :D // 2]
    hhi_ref[...] = h[:, D // 2:]
    as_ref[...] = _dot(h, As_ref[...])
    ad_ref[...] = _dot(h, Ad_ref[...])


def _prep(agg_lo, agg_hi, b_lo, b_hi, W_top, W_bot, As, Ad, R=2000):
    D = W_top.shape[1]
    grid = (N // R,)
    full = lambda a: pl.BlockSpec(a.shape, lambda i: (0,) * a.ndim)
    return pl.pallas_call(
        _prep_body,
        grid=grid,
        in_specs=[pl.BlockSpec((R, 128), lambda i: (i, 0)),
                  pl.BlockSpec((R, 128), lambda i: (i, 0)),
                  full(b_lo), full(b_hi), full(W_top), full(W_bot),
                  full(As), full(Ad)],
        out_specs=[pl.BlockSpec((R, D // 2), lambda i: (i, 0)),
                   pl.BlockSpec((R, D // 2), lambda i: (i, 0)),
                   pl.BlockSpec((R, 16), lambda i: (i, 0)),
                   pl.BlockSpec((R, 16), lambda i: (i, 0))],
        out_shape=[jax.ShapeDtypeStruct((N, D // 2), jnp.float32),
                   jax.ShapeDtypeStruct((N, D // 2), jnp.float32),
                   jax.ShapeDtypeStruct((N, 16), jnp.float32),
                   jax.ShapeDtypeStruct((N, 16), jnp.float32)],
    )(agg_lo, agg_hi, b_lo, b_hi, W_top, W_bot, As, Ad)


def _cpack_body(as_ref, ad_ref, adc_ref):
    gmax = jnp.max(as_ref[...], axis=0, keepdims=True)
    z = gmax + ad_ref[...]
    c = jnp.where(z >= 0.0, z, 0.2 * z)
    adc_ref[...] = jnp.concatenate([ad_ref[...], c], axis=1)


def _cpack(asrc, adst):
    return pl.pallas_call(
        _cpack_body,
        out_shape=jax.ShapeDtypeStruct((N, 32), jnp.float32),
    )(asrc, adst)


def _tail_body(p0_ref, p1_ref, g4b_ref, fc_W_ref, fc_b_ref, out_W_ref,
               out_b_ref, x_trunk_ref, t1_W_ref, t1_b_ref, t2_W_ref,
               t2_b_ref, t3_W_ref, t3_b_ref, t4_W_ref, t4_b_ref, bias_ref,
               o_ref):
    x4 = (p0_ref[...] + p1_ref[...]) * jnp.float32(1.0 / 16.0) + g4b_ref[...]
    x5 = _dot(x4, fc_W_ref[...]) + fc_b_ref[...]
    bout = _dot(x5, out_W_ref[...]) + out_b_ref[...]
    t = jax.nn.relu(_dot(x_trunk_ref[...], t1_W_ref[...]) + t1_b_ref[...])
    t = jax.nn.relu(_dot(t, t2_W_ref[...]) + t2_b_ref[...])
    t = jax.nn.relu(_dot(t, t3_W_ref[...]) + t3_b_ref[...])
    t = jax.nn.relu(_dot(t, t4_W_ref[...]) + t4_b_ref[...])
    o_ref[...] = _dot(bout, t.T) + bias_ref[0]


def _tail(p0, p1, g4_b, fc_W, fc_b, out_W, out_b, x_trunk,
          t1_W, t1_b, t2_W, t2_b, t3_W, t3_b, t4_W, t4_b, bias):
    T = x_trunk.shape[0]
    R = 2000
    grid = (N // R,)
    full = lambda a: pl.BlockSpec(a.shape, lambda i: (0,) * a.ndim)
    return pl.pallas_call(
        _tail_body,
        grid=grid,
        in_specs=[pl.BlockSpec((R, 128), lambda i: (i, 0)),
                  pl.BlockSpec((R, 128), lambda i: (i, 0)),
                  full(g4_b), full(fc_W), full(fc_b), full(out_W),
                  full(out_b), full(x_trunk), full(t1_W), full(t1_b),
                  full(t2_W), full(t2_b), full(t3_W), full(t3_b),
                  full(t4_W), full(t4_b), full(bias)],
        out_specs=pl.BlockSpec((R, T), lambda i: (i, 0)),
        out_shape=jax.ShapeDtypeStruct((N, T), jnp.float32),
    )(p0, p1, g4_b, fc_W, fc_b, out_W, out_b, x_trunk,
      t1_W, t1_b, t2_W, t2_b, t3_W, t3_b, t4_W, t4_b, bias)


def _expand_attn(a):
    H, C = a.shape
    return (a[:, :, None] * jnp.eye(H, dtype=a.dtype)[:, None, :]).reshape(
        H * C, H)


def _pad_adc(adc):
    pad = jnp.concatenate(
        [jnp.zeros((N_ACC - N, 16), jnp.float32),
         jnp.full((N_ACC - N, 16), 1e30, jnp.float32)], axis=1)
    return jnp.pad(jnp.concatenate([adc, pad]), ((0, 0), (0, 96)))


def _pad_lanes(a):
    return jnp.pad(a, ((0, 0), (0, 128 - a.shape[1])))


def _den128(den):
    return jnp.pad(jnp.concatenate([den[0], den[1]], axis=1),
                   ((0, 0), (0, 96)))


def kernel(x_branch, x_branch_index, x_trunk, g1_W, g1_as, g1_ad, g1_b,
           g2_W, g2_as, g2_ad, g2_b, g3_W, g3_as, g3_ad, g3_b,
           g4_W, g4_as, g4_ad, g4_b, fc_W, fc_b, out_W, out_b,
           t1_W, t1_b, t2_W, t2_b, t3_W, t3_b, t4_W, t4_b, bias):
    src = x_branch_index[0].astype(jnp.int32)
    dst = x_branch_index[1].astype(jnp.int32)
    dst_m = jnp.where(src == dst, TRASH, dst)
    loop = jnp.arange(N, dtype=jnp.int32)
    n_pad = E_PAD - EN
    src_all = jnp.concatenate([src, loop, jnp.zeros((n_pad,), jnp.int32)])
    dst_all = jnp.concatenate(
        [dst_m, loop, jnp.full((n_pad,), TRASH, jnp.int32)])

    x_pad = jnp.pad(x_branch, ((0, 0), (0, 3)))
    W1_pad = jnp.pad(g1_W, ((0, 3), (0, 0)))

    h_lo, h_hi, asrc, adst = _prep1(x_pad, W1_pad, _expand_attn(g1_as),
                                    _expand_attn(g1_ad))
    adc = _pad_adc(_cpack(asrc, adst))
    ex, den = _attn_pass(src_all, dst_all, _pad_lanes(asrc), adc)
    agg = _agg_pass(src_all, dst_all, ex, _den128(den), h_lo, h_hi)
    h_lo, h_hi, asrc, adst = _prep(agg[0][:N], agg[1][:N],
                                   g1_b[:128], g1_b[128:],
                                   g2_W[:128], g2_W[128:],
                                   _expand_attn(g2_as), _expand_attn(g2_ad))
    adc = _pad_adc(_cpack(asrc, adst))
    ex, den = _attn_pass(src_all, dst_all, _pad_lanes(asrc), adc)
    agg = _agg_pass(src_all, dst_all, ex, _den128(den), h_lo, h_hi)
    h_lo, h_hi, asrc, adst = _prep(agg[0][:N], agg[1][:N],
                                   g2_b[:128], g2_b[128:],
                                   g3_W[:128], g3_W[128:],
                                   _expand_attn(g3_as), _expand_attn(g3_ad))
    adc = _pad_adc(_cpack(asrc, adst))
    ex, den = _attn_pass(src_all, dst_all, _pad_lanes(asrc), adc)
    agg = _agg_pass(src_all, dst_all, ex, _den128(den), h_lo, h_hi)
    h_lo, h_hi, asrc, adst = _prep(agg[0][:N], agg[1][:N],
                                   g3_b[:128], g3_b[128:],
                                   g4_W[:128], g4_W[128:],
                                   _expand_attn(g4_as), _expand_attn(g4_ad),
                                   R=1000)
    adc = _pad_adc(_cpack(asrc, adst))
    ex, den = _attn_pass(src_all, dst_all, _pad_lanes(asrc), adc)
    p = _agg4_pass(src_all, dst_all, ex, _den128(den), h_lo, h_hi)
    return _tail(p[0][:N], p[1][:N], g4_b, fc_W, fc_b, out_W, out_b, x_trunk,
                 t1_W, t1_b, t2_W, t2_b, t3_W, t3_b, t4_W, t4_b, bias)
